# baseline (speedup 1.0000x reference)
import functools

import jax
import jax.numpy as jnp
from jax.experimental import pallas as pl
from jax.experimental.pallas import tpu as pltpu

K_CLUSTERS = 512
N_TOKENS = 16384
D_CODE = 64
MAX_ITERS = 100
BLK = 2048
NBLK = N_TOKENS // BLK
RTOL = 1e-4
ATOL = 1e-8


def _split3(v):
    hi = v.astype(jnp.bfloat16)
    r1 = v - hi.astype(jnp.float32)
    mid = r1.astype(jnp.bfloat16)
    lo = (r1 - mid.astype(jnp.float32)).astype(jnp.bfloat16)
    return hi, mid, lo


def _kmeans_loss_kernel(x_ref, ones_ref, xhi_ref, xmid_ref, xlo_ref, c0_ref,
                        out_ref, c_ref, sums_ref, counts_ref, ids_ref):
    iota_lane = jax.lax.broadcasted_iota(jnp.int32, (BLK, K_CLUSTERS), 1)
    ones64 = jnp.ones((1, D_CODE), jnp.bfloat16)
    c_ref[...] = c0_ref[...]

    def body(carry):
        it, _ = carry
        c = c_ref[...]
        b2_row = jnp.zeros((1, K_CLUSTERS), jnp.float32)
        for ccp in _split3(c * c):
            b2_row = b2_row + jax.lax.dot_general(
                ones64, ccp, (((1,), (1,)), ((), ())),
                preferred_element_type=jnp.float32)

        sums_ref[...] = jnp.zeros((K_CLUSTERS, D_CODE), jnp.float32)
        counts_ref[...] = jnp.zeros((K_CLUSTERS, 1), jnp.float32)
        for blk in range(NBLK):
            sl = pl.ds(blk * BLK, BLK)
            m = jax.lax.dot_general(x_ref[sl, :], c, (((1,), (1,)), ((), ())),
                                    preferred_element_type=jnp.float32)
            score = b2_row - 2.0 * m
            minval = jnp.min(score, axis=1, keepdims=True)
            ids = jnp.min(jnp.where(score == minval, iota_lane, K_CLUSTERS),
                          axis=1, keepdims=True)
            ids_ref[sl, :] = ids
            onehot = (iota_lane == ids).astype(jnp.bfloat16)
            acc = sums_ref[...]
            for xs_ref in (xhi_ref, xmid_ref, xlo_ref):
                acc = acc + jax.lax.dot_general(
                    onehot, xs_ref[sl, :], (((0,), (0,)), ((), ())),
                    preferred_element_type=jnp.float32)
            sums_ref[...] = acc
            counts_ref[...] += jax.lax.dot_general(
                onehot, ones_ref[sl, :], (((0,), (0,)), ((), ())),
                preferred_element_type=jnp.float32)

        counts = counts_ref[...]
        new_c = sums_ref[...] / jnp.maximum(counts, 1.0)
        new_c = jnp.where(counts > 0.0, new_c, c)
        ok = (jnp.abs(c - new_c) <= ATOL + RTOL * jnp.abs(new_c))
        converged = (jnp.min(ok.astype(jnp.float32)) >= 1.0).astype(jnp.int32)

        @pl.when(converged == 0)
        def _():
            c_ref[...] = new_c

        return it + 1, converged

    jax.lax.while_loop(
        lambda carry: jnp.logical_and(carry[0] < MAX_ITERS, carry[1] == 0),
        body, (jnp.int32(0), jnp.int32(0)))

    c = c_ref[...]
    total = jnp.float32(0.0)
    for blk in range(NBLK):
        sl = pl.ds(blk * BLK, BLK)
        m = jax.lax.dot_general(x_ref[sl, :], c, (((1,), (1,)), ((), ())),
                                preferred_element_type=jnp.float32)
        rowmax = jnp.max(m, axis=1, keepdims=True)
        lse = jnp.log(jnp.sum(jnp.exp(m - rowmax), axis=1,
                              keepdims=True)) + rowmax
        onehot = (iota_lane == ids_ref[sl, :]).astype(jnp.float32)
        label_logit = jnp.sum(m * onehot, axis=1, keepdims=True)
        total += jnp.sum(lse - label_logit)
    out_ref[0, 0] = total / jnp.float32(N_TOKENS)


@functools.partial(jax.jit, static_argnames=("interpret",))
def kernel(x, interpret=False):
    perm = jax.random.permutation(jax.random.key(42), N_TOKENS)
    c0 = x[perm[:K_CLUSTERS]]
    ones = jnp.ones((N_TOKENS, 1), jnp.bfloat16)
    x_hi, x_mid, x_lo = _split3(x)
    loss = pl.pallas_call(
        _kmeans_loss_kernel,
        out_shape=jax.ShapeDtypeStruct((1, 1), jnp.float32),
        in_specs=[pl.BlockSpec(memory_space=pltpu.VMEM)] * 6,
        out_specs=pl.BlockSpec(memory_space=pltpu.SMEM),
        scratch_shapes=[
            pltpu.VMEM((K_CLUSTERS, D_CODE), jnp.float32),
            pltpu.VMEM((K_CLUSTERS, D_CODE), jnp.float32),
            pltpu.VMEM((K_CLUSTERS, 1), jnp.float32),
            pltpu.VMEM((N_TOKENS, 1), jnp.int32),
        ],
        interpret=interpret,
    )(x, ones, x_hi, x_mid, x_lo, c0)
    return jnp.reshape(loss, ())


# speedup vs baseline: 1.0663x; 1.0663x over previous
import functools

import jax
import jax.numpy as jnp
from jax.experimental import pallas as pl
from jax.experimental.pallas import tpu as pltpu

K_CLUSTERS = 512
N_TOKENS = 16384
D_CODE = 64
D_STAT = D_CODE + 1
MAX_ITERS = 100
BLK = 2048
NBLK = N_TOKENS // BLK
RTOL = 1e-4
ATOL = 1e-8


def _kmeans_loss_kernel(x_ref, xhi_ref, xmid_ref, xlo_ref, c0_ref,
                        out_ref, c_ref, stats_ref, ids_ref):
    iota_sub = jax.lax.broadcasted_iota(jnp.int32, (K_CLUSTERS, BLK), 0)
    c_ref[...] = c0_ref[...]

    def body(carry):
        it, _ = carry
        c = c_ref[...]
        b2 = jnp.sum(c * c, axis=1, keepdims=True)

        stats_ref[...] = jnp.zeros((K_CLUSTERS, D_STAT), jnp.float32)
        for blk in range(NBLK):
            sl = pl.ds(blk * BLK, BLK)
            m = jax.lax.dot_general(c, x_ref[sl, :], (((1,), (1,)), ((), ())),
                                    preferred_element_type=jnp.float32)
            score = b2 - 2.0 * m
            minval = jnp.min(score, axis=0, keepdims=True)
            ids = jnp.min(jnp.where(score == minval, iota_sub, K_CLUSTERS),
                          axis=0, keepdims=True)
            ids_ref[blk:blk + 1, :] = ids
            onehot = (iota_sub == ids).astype(jnp.bfloat16)
            acc = stats_ref[...]
            for xs_ref in (xhi_ref, xmid_ref, xlo_ref):
                acc = acc + jax.lax.dot_general(
                    onehot, xs_ref[sl, :], (((1,), (0,)), ((), ())),
                    preferred_element_type=jnp.float32)
            stats_ref[...] = acc

        counts = stats_ref[:, D_CODE:D_STAT]
        new_c = stats_ref[:, :D_CODE] / jnp.maximum(counts, 1.0)
        new_c = jnp.where(counts > 0.0, new_c, c)
        ok = (jnp.abs(c - new_c) <= ATOL + RTOL * jnp.abs(new_c))
        converged = (jnp.min(ok.astype(jnp.float32)) >= 1.0).astype(jnp.int32)

        @pl.when(converged == 0)
        def _():
            c_ref[...] = new_c

        return it + 1, converged

    jax.lax.while_loop(
        lambda carry: jnp.logical_and(carry[0] < MAX_ITERS, carry[1] == 0),
        body, (jnp.int32(0), jnp.int32(0)))

    c = c_ref[...]
    total = jnp.float32(0.0)
    for blk in range(NBLK):
        sl = pl.ds(blk * BLK, BLK)
        m = jax.lax.dot_general(c, x_ref[sl, :], (((1,), (1,)), ((), ())),
                                preferred_element_type=jnp.float32)
        colmax = jnp.max(m, axis=0, keepdims=True)
        lse = jnp.log(jnp.sum(jnp.exp(m - colmax), axis=0,
                              keepdims=True)) + colmax
        onehot = (iota_sub == ids_ref[blk:blk + 1, :]).astype(jnp.float32)
        label_logit = jnp.sum(m * onehot, axis=0, keepdims=True)
        total += jnp.sum(lse - label_logit)
    out_ref[0, 0] = total / jnp.float32(N_TOKENS)


@functools.partial(jax.jit, static_argnames=("interpret",))
def kernel(x, interpret=False):
    perm = jax.random.permutation(jax.random.key(42), N_TOKENS)
    c0 = x[perm[:K_CLUSTERS]]
    x_hi = x.astype(jnp.bfloat16)
    r1 = x - x_hi.astype(jnp.float32)
    x_mid = r1.astype(jnp.bfloat16)
    x_lo = (r1 - x_mid.astype(jnp.float32)).astype(jnp.bfloat16)
    zcol = jnp.zeros((N_TOKENS, 1), jnp.bfloat16)
    ocol = jnp.ones((N_TOKENS, 1), jnp.bfloat16)
    x_hi = jnp.concatenate([x_hi, zcol], axis=1)
    x_mid = jnp.concatenate([x_mid, zcol], axis=1)
    x_lo = jnp.concatenate([x_lo, ocol], axis=1)
    loss = pl.pallas_call(
        _kmeans_loss_kernel,
        out_shape=jax.ShapeDtypeStruct((1, 1), jnp.float32),
        in_specs=[pl.BlockSpec(memory_space=pltpu.VMEM)] * 5,
        out_specs=pl.BlockSpec(memory_space=pltpu.SMEM),
        scratch_shapes=[
            pltpu.VMEM((K_CLUSTERS, D_CODE), jnp.float32),
            pltpu.VMEM((K_CLUSTERS, D_STAT), jnp.float32),
            pltpu.VMEM((NBLK, BLK), jnp.int32),
        ],
        interpret=interpret,
    )(x, x_hi, x_mid, x_lo, c0)
    return jnp.reshape(loss, ())
